# Initial kernel scaffold; baseline (speedup 1.0000x reference)
import jax
import jax.numpy as jnp
from jax.experimental import pallas as pl


def kernel(x, Wg, bg, w1, w2):
    raise NotImplementedError("write your pallas kernel here")


# speedup vs baseline: 1.9217x; 1.9217x over previous
import functools

import jax
import jax.numpy as jnp
from jax import lax
from jax.experimental import pallas as pl
from jax.experimental.pallas import tpu as pltpu
from jax.experimental.pallas import tpu_sc as plsc

E = 8
TOPK = 2
D = 1024
DFF = 2048
N = DFF // 2
T = 4096

BM = 256
P = T * TOPK + E * BM
NBLK = P // BM

NW = 32
TPW = T // NW
CH_D = 64
CH_C = 32
LANES = 16


def _routing_body(x_ref, wgt_ref, bg_ref, pos_ref, pw_ref, bexp_ref):
    x = x_ref[...]
    logits = jnp.dot(x, wgt_ref[...], preferred_element_type=jnp.float32)
    logits = logits + bg_ref[...]

    rE = lax.broadcasted_iota(jnp.int32, (E, E), 0)
    cE = lax.broadcasted_iota(jnp.int32, (E, E), 1)
    strictE = (rE < cE).astype(jnp.float32)

    def first_max_onehot(l):
        m = jnp.max(l, axis=1, keepdims=True)
        eq = (l == m).astype(jnp.float32)
        cume = jnp.dot(eq, strictE, preferred_element_type=jnp.float32)
        oh = eq * (cume == 0.0).astype(jnp.float32)
        return oh, m

    oh1, m1 = first_max_onehot(logits)
    oh2, m2 = first_max_onehot(logits - oh1 * 1e30)
    w_a = 1.0 / (1.0 + jnp.exp(m2 - m1))
    w_b = 1.0 - w_a

    ohcat = jnp.concatenate([oh1, oh2], axis=1)
    CH = 256
    rC = lax.broadcasted_iota(jnp.int32, (CH, CH), 0)
    cC = lax.broadcasted_iota(jnp.int32, (CH, CH), 1)
    strictC = (rC > cC).astype(jnp.float32)
    carry = jnp.zeros((1, 2 * E), jnp.float32)
    chunks = []
    for ci in range(T // CH):
        blk = lax.slice(ohcat, (ci * CH, 0), ((ci + 1) * CH, 2 * E))
        chunks.append(jnp.dot(strictC, blk, preferred_element_type=jnp.float32) + carry)
        carry = carry + jnp.sum(blk, axis=0, keepdims=True)
    cume = jnp.concatenate(chunks, axis=0)
    cnt1 = carry[:, :E]
    cnt = cnt1 + carry[:, E:]
    cnt_al = jnp.ceil(cnt / BM) * BM
    offs = jnp.dot(cnt_al, strictE, preferred_element_type=jnp.float32)
    incl = offs + cnt_al

    pos0 = jnp.sum(oh1 * (offs + cume[:, :E]), axis=1, keepdims=True)
    pos1 = jnp.sum(oh2 * (offs + cnt1 + cume[:, E:]), axis=1, keepdims=True)
    pos_ref[...] = jnp.concatenate([pos0, pos1], axis=1).astype(jnp.int32)
    pw_ref[...] = jnp.concatenate([w_a, w_b], axis=1)

    bstart = lax.broadcasted_iota(jnp.int32, (128, 1), 0).astype(jnp.float32) * BM
    nle = jnp.sum((bstart >= incl).astype(jnp.float32), axis=1, keepdims=True)
    bexp = jnp.minimum(nle, float(E - 1)).astype(jnp.int32)
    bexp_ref[...] = jax.lax.broadcast_in_dim(bexp, (128, E), (0, 1))


def _routing(x2d, wg_t, bg_row, interpret=False):
    return pl.pallas_call(
        _routing_body,
        out_shape=(
            jax.ShapeDtypeStruct((T, TOPK), jnp.int32),
            jax.ShapeDtypeStruct((T, TOPK), jnp.float32),
            jax.ShapeDtypeStruct((128, E), jnp.int32),
        ),
        interpret=interpret,
    )(x2d, wg_t, bg_row)


def _dispatch_body(x_hbm, pos0_hbm, pos1_hbm, xs_hbm, idx0_v, idx1_v, rows_v, sem):
    wid = lax.axis_index("s") * 2 + lax.axis_index("c")
    t0 = wid * TPW
    pltpu.sync_copy(pos0_hbm.at[wid], idx0_v)
    pltpu.sync_copy(pos1_hbm.at[wid], idx1_v)
    for j in range(TPW // CH_D):
        pltpu.sync_copy(x_hbm.at[pl.ds(t0 + j * CH_D, CH_D)], rows_v)
        pltpu.async_copy(rows_v, xs_hbm.at[idx0_v.at[j]], sem).wait()
        pltpu.async_copy(rows_v, xs_hbm.at[idx1_v.at[j]], sem).wait()


def _dispatch(x2d, pos0, pos1, interpret=False):
    mesh = plsc.VectorSubcoreMesh(core_axis_name="c", subcore_axis_name="s")
    kfn = functools.partial(
        pl.kernel,
        out_type=jax.ShapeDtypeStruct((P, D), jnp.float32),
        mesh=mesh,
        scratch_types=[
            pltpu.VMEM((TPW // CH_D, CH_D), jnp.int32),
            pltpu.VMEM((TPW // CH_D, CH_D), jnp.int32),
            pltpu.VMEM((CH_D, D), jnp.float32),
            pltpu.SemaphoreType.DMA,
        ],
        interpret=interpret,
    )(_dispatch_body)
    return kfn(x2d, pos0, pos1)


def _ffn_body(bexp_ref, xs_ref, w1_ref, w2_ref, ys_ref):
    xb = xs_ref[...].astype(jnp.bfloat16)
    w1e = w1_ref[0].astype(jnp.bfloat16)
    h = lax.dot_general(xb, w1e, (((1,), (1,)), ((), ())),
                        preferred_element_type=jnp.float32)
    g = h[:, :N]
    u = h[:, N:]
    act = g * (1.0 / (1.0 + jnp.exp(-g))) * u
    w2e = w2_ref[0].astype(jnp.bfloat16)
    y = lax.dot_general(act.astype(jnp.bfloat16), w2e,
                        (((1,), (1,)), ((), ())),
                        preferred_element_type=jnp.float32)
    ys_ref[...] = y


def _ffn(bexp, xs, w1, w2, interpret=False):
    grid_spec = pltpu.PrefetchScalarGridSpec(
        num_scalar_prefetch=1,
        grid=(NBLK,),
        in_specs=[
            pl.BlockSpec((BM, D), lambda i, s: (i, 0)),
            pl.BlockSpec((1, DFF, D), lambda i, s: (s[i], 0, 0)),
            pl.BlockSpec((1, D, N), lambda i, s: (s[i], 0, 0)),
        ],
        out_specs=pl.BlockSpec((BM, D), lambda i, s: (i, 0)),
    )
    return pl.pallas_call(
        _ffn_body,
        grid_spec=grid_spec,
        out_shape=jax.ShapeDtypeStruct((P, D), jnp.float32),
        interpret=interpret,
    )(bexp, xs, w1, w2)


def _combine_body(ys_hbm, pos0_hbm, pos1_hbm, w0_hbm, w1_hbm, out_hbm,
                  idx0_v, idx1_v, w0_v, w1_v, a_v, b_v, sem):
    wid = lax.axis_index("s") * 2 + lax.axis_index("c")
    t0 = wid * TPW
    pltpu.sync_copy(pos0_hbm.at[wid], idx0_v)
    pltpu.sync_copy(pos1_hbm.at[wid], idx1_v)
    pltpu.sync_copy(w0_hbm.at[wid], w0_v)
    pltpu.sync_copy(w1_hbm.at[wid], w1_v)
    for j in range(TPW // CH_C):
        pltpu.async_copy(ys_hbm.at[idx0_v.at[j]], a_v, sem).wait()
        pltpu.async_copy(ys_hbm.at[idx1_v.at[j]], b_v, sem).wait()

        def row_body(r, _):
            tok = j * CH_C + r
            wa = w0_v[tok, :]
            wb = w1_v[tok, :]

            def vec_body(v, _):
                sl = pl.ds(v * LANES, LANES)
                a_v[r, sl] = a_v[r, sl] * wa + b_v[r, sl] * wb
                return 0

            return lax.fori_loop(0, D // LANES, vec_body, 0)

        lax.fori_loop(0, CH_C, row_body, 0)
        pltpu.sync_copy(a_v, out_hbm.at[pl.ds(t0 + j * CH_C, CH_C)])


def _combine(ys, pos0, pos1, w0, w1, interpret=False):
    mesh = plsc.VectorSubcoreMesh(core_axis_name="c", subcore_axis_name="s")
    kfn = functools.partial(
        pl.kernel,
        out_type=jax.ShapeDtypeStruct((T, D), jnp.float32),
        mesh=mesh,
        scratch_types=[
            pltpu.VMEM((TPW // CH_C, CH_C), jnp.int32),
            pltpu.VMEM((TPW // CH_C, CH_C), jnp.int32),
            pltpu.VMEM((TPW, LANES), jnp.float32),
            pltpu.VMEM((TPW, LANES), jnp.float32),
            pltpu.VMEM((CH_C, D), jnp.float32),
            pltpu.VMEM((CH_C, D), jnp.float32),
            pltpu.SemaphoreType.DMA,
        ],
        interpret=interpret,
    )(_combine_body)
    return kfn(ys, pos0, pos1, w0, w1)


def kernel(x, Wg, bg, w1, w2):
    Bs, Ss, _ = x.shape
    x2d = x.reshape(T, D)

    pos, pw, bexp2d = _routing(x2d, Wg.T, bg.reshape(1, E))
    bexp = bexp2d[:NBLK, 0]

    pos0 = pos[:, 0].reshape(NW, TPW // CH_D, CH_D)
    pos1 = pos[:, 1].reshape(NW, TPW // CH_D, CH_D)
    xs = _dispatch(x2d, pos0, pos1)

    ys = _ffn(bexp, xs, w1, w2)

    pos0c = pos[:, 0].reshape(NW, TPW // CH_C, CH_C)
    pos1c = pos[:, 1].reshape(NW, TPW // CH_C, CH_C)
    w0e = jnp.broadcast_to(pw[:, 0].reshape(NW, TPW, 1), (NW, TPW, LANES))
    w1e = jnp.broadcast_to(pw[:, 1].reshape(NW, TPW, 1), (NW, TPW, LANES))
    out = _combine(ys, pos0c, pos1c, w0e, w1e)
    return out.reshape(Bs, Ss, D)
